# Initial kernel scaffold; baseline (speedup 1.0000x reference)
import jax
import jax.numpy as jnp
from jax.experimental import pallas as pl


def kernel(x, cW1, cb1, cg, cbb, cW2, cb2, eW1, eb1, eg, ebb, eW2, eb2, aW1, ab1, ag, abb, aW2, ab2):
    raise NotImplementedError("write your pallas kernel here")


# speedup vs baseline: 1.9126x; 1.9126x over previous
import functools

import jax
import jax.numpy as jnp
from jax.experimental import pallas as pl
from jax.experimental.pallas import tpu as pltpu

_TM = 256
_LANES = 128
_NEG = -1e30


def _gelu(h):
    return 0.5 * h * (1.0 + jax.lax.erf(h * 0.7071067811865476))


def _ln_f32(h, g, b):
    m = jnp.mean(h, axis=-1, keepdims=True)
    d = h - m
    v = jnp.mean(d * d, axis=-1, keepdims=True)
    return d * jax.lax.rsqrt(v + 1e-5) * g + b


def _top2(p):
    idx = jax.lax.broadcasted_iota(jnp.int32, p.shape, 1)
    w1 = jnp.max(p, axis=-1, keepdims=True)
    i1 = jnp.min(jnp.where(p == w1, idx, jnp.int32(2**30)), axis=-1, keepdims=True)
    p2 = jnp.where(idx == i1, _NEG, p)
    w2 = jnp.max(p2, axis=-1, keepdims=True)
    i2 = jnp.min(jnp.where(p2 == w2, idx, jnp.int32(2**30)), axis=-1, keepdims=True)
    return w1, i1, w2, i2


def _softmax_masked(l):
    m = jnp.max(l, axis=-1, keepdims=True)
    e = jnp.exp(l - m)
    return e / jnp.sum(e, axis=-1, keepdims=True)


def _router_body(nclusters,
                 xb_ref, cW1_ref, cb1_ref, cW2_ref, cb2_ref, cg_ref, cbb_ref,
                 eW1_ref, eW2_ref, eb2_ref, eb1_ref, eg_ref, ebb_ref,
                 aW1_ref, aW2_ref, ab2_ref, ab1_ref, ag_ref, abb_ref,
                 cw_ref, ci_ref, ew0_ref, ei0_ref, ew1_ref, ei1_ref, ad_ref):
    xb = xb_ref[...]

    h = jnp.dot(xb, cW1_ref[...], preferred_element_type=jnp.float32)
    h = _ln_f32(h + cb1_ref[...], cg_ref[...], cbb_ref[...])
    h = _gelu(h)
    cl = jnp.dot(h.astype(jnp.bfloat16), cW2_ref[...],
                 preferred_element_type=jnp.float32) + cb2_ref[...]
    cp = _softmax_masked(cl)
    cw1, ci1, cw2, ci2 = _top2(cp)

    sel0 = jnp.full(cl.shape, _NEG, dtype=jnp.float32)
    sel1 = jnp.full(cl.shape, _NEG, dtype=jnp.float32)
    for c in range(nclusters):
        eh = jnp.dot(xb, eW1_ref[c], preferred_element_type=jnp.float32)
        eh = _ln_f32(eh + eb1_ref[c], eg_ref[c], ebb_ref[c])
        eh = _gelu(eh)
        el = jnp.dot(eh.astype(jnp.bfloat16), eW2_ref[c],
                     preferred_element_type=jnp.float32) + eb2_ref[c]
        sel0 = jnp.where(ci1 == c, el, sel0)
        sel1 = jnp.where(ci2 == c, el, sel1)

    p0 = _softmax_masked(sel0)
    p1 = _softmax_masked(sel1)
    e0w1, e0i1, e0w2, e0i2 = _top2(p0)
    e1w1, e1i1, e1w2, e1i2 = _top2(p1)

    ah = jnp.dot(xb, aW1_ref[...], preferred_element_type=jnp.float32)
    ah = _ln_f32(ah + ab1_ref[...], ag_ref[...], abb_ref[...])
    ah = _gelu(ah)
    av = jnp.dot(ah.astype(jnp.bfloat16), aW2_ref[...],
                 preferred_element_type=jnp.float32) + ab2_ref[...]
    ad = jax.nn.sigmoid(av[:, 0:1])

    cw_ref[...] = jnp.concatenate([cw1, cw2], axis=1)
    ci_ref[...] = jnp.concatenate([ci1, ci2], axis=1)
    ew0_ref[...] = jnp.concatenate([e0w1, e0w2], axis=1)
    ei0_ref[...] = jnp.concatenate([e0i1, e0i2], axis=1)
    ew1_ref[...] = jnp.concatenate([e1w1, e1w2], axis=1)
    ei1_ref[...] = jnp.concatenate([e1i1, e1i2], axis=1)
    ad_ref[...] = ad


def kernel(x, cW1, cb1, cg, cbb, cW2, cb2, eW1, eb1, eg, ebb, eW2, eb2,
           aW1, ab1, ag, abb, aW2, ab2):
    Bz, Sz, Dz = x.shape
    T = Bz * Sz
    C = eW1.shape[0]
    H = cW1.shape[1]
    AH = aW1.shape[1]
    L = _LANES

    xb = x.reshape(T, Dz).astype(jnp.bfloat16)

    cW1b = cW1.astype(jnp.bfloat16)
    cW2p = jnp.zeros((H, L), cW2.dtype).at[:, :C].set(cW2).astype(jnp.bfloat16)
    cb2p = jnp.full((L,), _NEG, jnp.float32).at[:C].set(cb2)
    eW1b = eW1.astype(jnp.bfloat16)
    EPC = eW2.shape[2]
    eW2p = jnp.zeros((C, H, L), eW2.dtype).at[:, :, :EPC].set(eW2).astype(jnp.bfloat16)
    eb2p = jnp.full((C, L), _NEG, jnp.float32).at[:, :EPC].set(eb2)
    aW1b = aW1.astype(jnp.bfloat16)
    aW2p = jnp.zeros((AH, L), aW2.dtype).at[:, :1].set(aW2).astype(jnp.bfloat16)
    ab2p = jnp.zeros((L,), jnp.float32).at[:1].set(ab2)

    grid = (T // _TM,)
    tok_spec = pl.BlockSpec((_TM, Dz), lambda i: (i, 0))
    full = lambda *shape: pl.BlockSpec(shape, lambda i: (0,) * len(shape))
    out2 = pl.BlockSpec((_TM, 2), lambda i: (i, 0))
    out1 = pl.BlockSpec((_TM, 1), lambda i: (i, 0))

    f32 = jnp.float32
    i32 = jnp.int32
    outs = (
        jax.ShapeDtypeStruct((T, 2), f32), jax.ShapeDtypeStruct((T, 2), i32),
        jax.ShapeDtypeStruct((T, 2), f32), jax.ShapeDtypeStruct((T, 2), i32),
        jax.ShapeDtypeStruct((T, 2), f32), jax.ShapeDtypeStruct((T, 2), i32),
        jax.ShapeDtypeStruct((T, 1), f32),
    )

    cw, ci, ew0, ei0, ew1, ei1, ad = pl.pallas_call(
        functools.partial(_router_body, C),
        grid=grid,
        in_specs=[
            tok_spec,
            full(Dz, H), full(H), full(H, L), full(L), full(H), full(H),
            full(C, Dz, H), full(C, H, L), full(C, L), full(C, H),
            full(C, H), full(C, H),
            full(Dz, AH), full(AH, L), full(L), full(AH), full(AH), full(AH),
        ],
        out_specs=[out2, out2, out2, out2, out2, out2, out1],
        out_shape=outs,
        compiler_params=pltpu.CompilerParams(
            dimension_semantics=("arbitrary",),
        ),
    )(xb, cW1b, cb1, cW2p, cb2p, cg, cbb,
      eW1b, eW2p, eb2p, eb1, eg, ebb,
      aW1b, aW2p, ab2p, ab1, ag, abb)

    return (cw, ci, ew0, ei0, ew1, ei1, ad.reshape(Bz, Sz, 1))


# speedup vs baseline: 1.9785x; 1.0345x over previous
import functools

import jax
import jax.numpy as jnp
from jax.experimental import pallas as pl
from jax.experimental.pallas import tpu as pltpu

_TM = 512
_LANES = 128
_GRP = 16
_NEG = -1e30


def _gelu(h):
    return 0.5 * h * (1.0 + jax.lax.erf(h * 0.7071067811865476))


def _ln_f32(h, g, b):
    m = jnp.mean(h, axis=-1, keepdims=True)
    d = h - m
    v = jnp.mean(d * d, axis=-1, keepdims=True)
    return d * jax.lax.rsqrt(v + 1e-5) * g + b


def _top2(p):
    idx = jax.lax.broadcasted_iota(jnp.int32, p.shape, 1)
    w1 = jnp.max(p, axis=-1, keepdims=True)
    i1 = jnp.min(jnp.where(p == w1, idx, jnp.int32(2**30)), axis=-1, keepdims=True)
    p2 = jnp.where(idx == i1, _NEG, p)
    w2 = jnp.max(p2, axis=-1, keepdims=True)
    i2 = jnp.min(jnp.where(p2 == w2, idx, jnp.int32(2**30)), axis=-1, keepdims=True)
    return w1, i1, w2, i2


def _softmax_masked(l):
    m = jnp.max(l, axis=-1, keepdims=True)
    e = jnp.exp(l - m)
    return e / jnp.sum(e, axis=-1, keepdims=True)


def _router_body(C, H, AH,
                 xb_ref, W1_ref, cb1_ref, cW2_ref, cb2_ref, cg_ref, cbb_ref,
                 W2e_ref, eb2_ref, eb1_ref, eg_ref, ebb_ref,
                 aW2_ref, ab2_ref, ab1_ref, ag_ref, abb_ref,
                 cw_ref, ci_ref, ew0_ref, ei0_ref, ew1_ref, ei1_ref, ad_ref):
    xb = xb_ref[...]

    mm = jnp.dot(xb, W1_ref[...], preferred_element_type=jnp.float32)

    h = _gelu(_ln_f32(mm[:, :H] + cb1_ref[...], cg_ref[...], cbb_ref[...]))
    cl = jnp.dot(h.astype(jnp.bfloat16), cW2_ref[...],
                 preferred_element_type=jnp.float32) + cb2_ref[...]
    cp = _softmax_masked(cl)
    cw1, ci1, cw2, ci2 = _top2(cp)

    ehb = []
    for c in range(C):
        seg = mm[:, H + c * H:H + (c + 1) * H]
        eh = _gelu(_ln_f32(seg + eb1_ref[c], eg_ref[c], ebb_ref[c]))
        ehb.append(eh.astype(jnp.bfloat16))
    ehb = jnp.concatenate(ehb, axis=1)
    el = jnp.dot(ehb, W2e_ref[...],
                 preferred_element_type=jnp.float32) + eb2_ref[...]

    sel0 = jnp.full((xb.shape[0], _GRP), _NEG, dtype=jnp.float32)
    sel1 = jnp.full((xb.shape[0], _GRP), _NEG, dtype=jnp.float32)
    for c in range(C):
        grp = el[:, c * _GRP:(c + 1) * _GRP]
        sel0 = jnp.where(ci1 == c, grp, sel0)
        sel1 = jnp.where(ci2 == c, grp, sel1)
    e0w1, e0i1, e0w2, e0i2 = _top2(_softmax_masked(sel0))
    e1w1, e1i1, e1w2, e1i2 = _top2(_softmax_masked(sel1))

    ah = _gelu(_ln_f32(mm[:, H + C * H:H + C * H + AH] + ab1_ref[...],
                       ag_ref[...], abb_ref[...]))
    av = jnp.dot(ah.astype(jnp.bfloat16), aW2_ref[...],
                 preferred_element_type=jnp.float32) + ab2_ref[...]
    ad = jax.nn.sigmoid(av[:, 0:1])

    cw_ref[...] = jnp.concatenate([cw1, cw2], axis=1)
    ci_ref[...] = jnp.concatenate([ci1, ci2], axis=1)
    ew0_ref[...] = jnp.concatenate([e0w1, e0w2], axis=1)
    ei0_ref[...] = jnp.concatenate([e0i1, e0i2], axis=1)
    ew1_ref[...] = jnp.concatenate([e1w1, e1w2], axis=1)
    ei1_ref[...] = jnp.concatenate([e1i1, e1i2], axis=1)
    ad_ref[...] = ad


def kernel(x, cW1, cb1, cg, cbb, cW2, cb2, eW1, eb1, eg, ebb, eW2, eb2,
           aW1, ab1, ag, abb, aW2, ab2):
    Bz, Sz, Dz = x.shape
    T = Bz * Sz
    C = eW1.shape[0]
    H = cW1.shape[1]
    AH = aW1.shape[1]
    EPC = eW2.shape[2]
    L = _LANES

    xb = x.reshape(T, Dz).astype(jnp.bfloat16)

    W1 = jnp.concatenate(
        [cW1] + [eW1[c] for c in range(C)] + [aW1], axis=1).astype(jnp.bfloat16)

    cW2p = jnp.zeros((H, L), cW2.dtype).at[:, :C].set(cW2).astype(jnp.bfloat16)
    cb2p = jnp.full((L,), _NEG, jnp.float32).at[:C].set(cb2)
    W2e = jnp.zeros((C * H, L), eW2.dtype)
    eb2p = jnp.full((L,), _NEG, jnp.float32)
    for c in range(C):
        W2e = W2e.at[c * H:(c + 1) * H, c * _GRP:c * _GRP + EPC].set(eW2[c])
        eb2p = eb2p.at[c * _GRP:c * _GRP + EPC].set(eb2[c])
    W2e = W2e.astype(jnp.bfloat16)
    aW2p = jnp.zeros((AH, L), aW2.dtype).at[:, :1].set(aW2).astype(jnp.bfloat16)
    ab2p = jnp.zeros((L,), jnp.float32).at[:1].set(ab2)

    grid = (T // _TM,)
    tok_spec = pl.BlockSpec((_TM, Dz), lambda i: (i, 0))
    full = lambda *shape: pl.BlockSpec(shape, lambda i: (0,) * len(shape))
    out2 = pl.BlockSpec((_TM, 2), lambda i: (i, 0))
    out1 = pl.BlockSpec((_TM, 1), lambda i: (i, 0))

    f32 = jnp.float32
    i32 = jnp.int32
    outs = (
        jax.ShapeDtypeStruct((T, 2), f32), jax.ShapeDtypeStruct((T, 2), i32),
        jax.ShapeDtypeStruct((T, 2), f32), jax.ShapeDtypeStruct((T, 2), i32),
        jax.ShapeDtypeStruct((T, 2), f32), jax.ShapeDtypeStruct((T, 2), i32),
        jax.ShapeDtypeStruct((T, 1), f32),
    )

    cw, ci, ew0, ei0, ew1, ei1, ad = pl.pallas_call(
        functools.partial(_router_body, C, H, AH),
        grid=grid,
        in_specs=[
            tok_spec,
            full(Dz, H + C * H + AH), full(H), full(H, L), full(L),
            full(H), full(H),
            full(C * H, L), full(L), full(C, H), full(C, H), full(C, H),
            full(AH, L), full(L), full(AH), full(AH), full(AH),
        ],
        out_specs=[out2, out2, out2, out2, out2, out2, out1],
        out_shape=outs,
        compiler_params=pltpu.CompilerParams(
            dimension_semantics=("arbitrary",),
        ),
    )(xb, W1, cb1, cW2p, cb2p, cg, cbb,
      W2e, eb2p, eb1, eg, ebb,
      aW2p, ab2p, ab1, ag, abb)

    return (cw, ci, ew0, ei0, ew1, ei1, ad.reshape(Bz, Sz, 1))


# speedup vs baseline: 2.8233x; 1.4269x over previous
import functools

import jax
import jax.numpy as jnp
from jax.experimental import pallas as pl
from jax.experimental.pallas import tpu as pltpu

_TM = 512
_LANES = 128
_GRP = 16
_NEG = -1e30


def _gelu(h):
    return 0.5 * h * (1.0 + jax.lax.erf(h * 0.7071067811865476))


def _ln_f32(h, g, b):
    m = jnp.mean(h, axis=-1, keepdims=True)
    d = h - m
    v = jnp.mean(d * d, axis=-1, keepdims=True)
    return d * jax.lax.rsqrt(v + 1e-5) * g + b


def _top2_t(p):
    idx = jax.lax.broadcasted_iota(jnp.int32, p.shape, 0)
    w1 = jnp.max(p, axis=0, keepdims=True)
    i1 = jnp.min(jnp.where(p == w1, idx, jnp.int32(2**30)), axis=0, keepdims=True)
    p2 = jnp.where(idx == i1, _NEG, p)
    w2 = jnp.max(p2, axis=0, keepdims=True)
    i2 = jnp.min(jnp.where(p2 == w2, idx, jnp.int32(2**30)), axis=0, keepdims=True)
    return w1, i1, w2, i2


def _softmax_t(l):
    m = jnp.max(l, axis=0, keepdims=True)
    e = jnp.exp(l - m)
    return e / jnp.sum(e, axis=0, keepdims=True)


def _router_body(C, H, AH,
                 xb_ref, W1_ref, cb1_ref, cW2_ref, cb2_ref, cg_ref, cbb_ref,
                 W2e_ref, eb2_ref, eb1_ref, eg_ref, ebb_ref,
                 aW2_ref, ab2_ref, ab1_ref, ag_ref, abb_ref,
                 cw_ref, ci_ref, ew0_ref, ei0_ref, ew1_ref, ei1_ref, ad_ref):
    xb = xb_ref[...]

    mm = jnp.dot(xb, W1_ref[...], preferred_element_type=jnp.float32)

    h = _gelu(_ln_f32(mm[:, :H] + cb1_ref[...], cg_ref[...], cbb_ref[...]))
    cl = jnp.dot(h.astype(jnp.bfloat16), cW2_ref[...],
                 preferred_element_type=jnp.float32) + cb2_ref[...]
    clT = jnp.transpose(cl, (1, 0))[:C]
    cp = _softmax_t(clT)
    cw1, ci1, cw2, ci2 = _top2_t(cp)

    ehb = []
    for c in range(C):
        seg = mm[:, H + c * H:H + (c + 1) * H]
        eh = _gelu(_ln_f32(seg + eb1_ref[c], eg_ref[c], ebb_ref[c]))
        ehb.append(eh.astype(jnp.bfloat16))
    ehb = jnp.concatenate(ehb, axis=1)
    el = jnp.dot(ehb, W2e_ref[...],
                 preferred_element_type=jnp.float32) + eb2_ref[...]
    elT = jnp.transpose(el, (1, 0))

    nE = elT.shape[0] // C
    sel0 = jnp.full((nE, elT.shape[1]), _NEG, dtype=jnp.float32)
    sel1 = jnp.full((nE, elT.shape[1]), _NEG, dtype=jnp.float32)
    for c in range(C):
        grp = elT[c * nE:(c + 1) * nE]
        sel0 = jnp.where(ci1 == c, grp, sel0)
        sel1 = jnp.where(ci2 == c, grp, sel1)
    e0w1, e0i1, e0w2, e0i2 = _top2_t(_softmax_t(sel0))
    e1w1, e1i1, e1w2, e1i2 = _top2_t(_softmax_t(sel1))

    ah = _gelu(_ln_f32(mm[:, H + C * H:H + C * H + AH] + ab1_ref[...],
                       ag_ref[...], abb_ref[...]))
    av = jnp.dot(ah.astype(jnp.bfloat16), aW2_ref[...],
                 preferred_element_type=jnp.float32) + ab2_ref[...]
    ad = jax.nn.sigmoid(av[:, 0:1])

    cw_ref[...] = jnp.concatenate([cw1, cw2], axis=0)
    ci_ref[...] = jnp.concatenate([ci1, ci2], axis=0)
    ew0_ref[...] = jnp.concatenate([e0w1, e0w2], axis=0)
    ei0_ref[...] = jnp.concatenate([e0i1, e0i2], axis=0)
    ew1_ref[...] = jnp.concatenate([e1w1, e1w2], axis=0)
    ei1_ref[...] = jnp.concatenate([e1i1, e1i2], axis=0)
    ad_ref[...] = ad


def kernel(x, cW1, cb1, cg, cbb, cW2, cb2, eW1, eb1, eg, ebb, eW2, eb2,
           aW1, ab1, ag, abb, aW2, ab2):
    Bz, Sz, Dz = x.shape
    T = Bz * Sz
    C = eW1.shape[0]
    H = cW1.shape[1]
    AH = aW1.shape[1]
    EPC = eW2.shape[2]
    L = _LANES

    xb = x.reshape(T, Dz).astype(jnp.bfloat16)

    W1 = jnp.concatenate(
        [cW1] + [eW1[c] for c in range(C)] + [aW1], axis=1).astype(jnp.bfloat16)

    cW2p = jnp.zeros((H, L), cW2.dtype).at[:, :C].set(cW2).astype(jnp.bfloat16)
    cb2p = jnp.full((L,), _NEG, jnp.float32).at[:C].set(cb2)
    W2e = jnp.zeros((C * H, L), eW2.dtype)
    eb2p = jnp.full((L,), _NEG, jnp.float32)
    for c in range(C):
        W2e = W2e.at[c * H:(c + 1) * H, c * _GRP:c * _GRP + EPC].set(eW2[c])
        eb2p = eb2p.at[c * _GRP:c * _GRP + EPC].set(eb2[c])
    W2e = W2e.astype(jnp.bfloat16)
    aW2p = jnp.zeros((AH, L), aW2.dtype).at[:, :1].set(aW2).astype(jnp.bfloat16)
    ab2p = jnp.zeros((L,), jnp.float32).at[:1].set(ab2)

    grid = (T // _TM,)
    tok_spec = pl.BlockSpec((_TM, Dz), lambda i: (i, 0))
    full = lambda *shape: pl.BlockSpec(shape, lambda i: (0,) * len(shape))
    out2 = pl.BlockSpec((2, _TM), lambda i: (0, i))
    out1 = pl.BlockSpec((_TM, 1), lambda i: (i, 0))

    f32 = jnp.float32
    i32 = jnp.int32
    outs = (
        jax.ShapeDtypeStruct((2, T), f32), jax.ShapeDtypeStruct((2, T), i32),
        jax.ShapeDtypeStruct((2, T), f32), jax.ShapeDtypeStruct((2, T), i32),
        jax.ShapeDtypeStruct((2, T), f32), jax.ShapeDtypeStruct((2, T), i32),
        jax.ShapeDtypeStruct((T, 1), f32),
    )

    cw, ci, ew0, ei0, ew1, ei1, ad = pl.pallas_call(
        functools.partial(_router_body, C, H, AH),
        grid=grid,
        in_specs=[
            tok_spec,
            full(Dz, H + C * H + AH), full(H), full(H, L), full(L),
            full(H), full(H),
            full(C * H, L), full(L), full(C, H), full(C, H), full(C, H),
            full(AH, L), full(L), full(AH), full(AH), full(AH),
        ],
        out_specs=[out2, out2, out2, out2, out2, out2, out1],
        out_shape=outs,
        compiler_params=pltpu.CompilerParams(
            dimension_semantics=("arbitrary",),
        ),
    )(xb, W1, cb1, cW2p, cb2p, cg, cbb,
      W2e, eb2p, eb1, eg, ebb,
      aW2p, ab2p, ab1, ag, abb)

    return (cw.T, ci.T, ew0.T, ei0.T, ew1.T, ei1.T, ad.reshape(Bz, Sz, 1))


# speedup vs baseline: 2.8696x; 1.0164x over previous
import functools

import jax
import jax.numpy as jnp
from jax.experimental import pallas as pl
from jax.experimental.pallas import tpu as pltpu

_TM = 512
_LANES = 128
_GRP = 16
_NEG = -1e30


def _gelu(h):
    return 0.5 * h * (1.0 + jax.lax.erf(h * 0.7071067811865476))


def _ln_f32(h, g, b):
    m = jnp.mean(h, axis=-1, keepdims=True)
    d = h - m
    v = jnp.mean(d * d, axis=-1, keepdims=True)
    return d * jax.lax.rsqrt(v + 1e-5) * g + b


def _top2_t(p):
    idx = jax.lax.broadcasted_iota(jnp.int32, p.shape, 0).astype(jnp.float32)
    w1 = jnp.max(p, axis=0, keepdims=True)
    i1 = jnp.min(jnp.where(p == w1, idx, jnp.float32(1e9)), axis=0, keepdims=True)
    p2 = jnp.where(idx == i1, _NEG, p)
    w2 = jnp.max(p2, axis=0, keepdims=True)
    i2 = jnp.min(jnp.where(p2 == w2, idx, jnp.float32(1e9)), axis=0, keepdims=True)
    return w1, i1, w2, i2


def _softmax_t(l):
    m = jnp.max(l, axis=0, keepdims=True)
    e = jnp.exp(l - m)
    return e / jnp.sum(e, axis=0, keepdims=True)


def _pair_out(a, b):
    return jnp.transpose(jnp.concatenate([a, b], axis=0), (1, 0))


def _router_body(C, H, AH,
                 x_ref, cW1_ref, eW1_ref, aW1_ref,
                 cb1_ref, cW2_ref, cb2_ref, cg_ref, cbb_ref,
                 W2e_ref, eb2_ref, eb1_ref, eg_ref, ebb_ref,
                 aW2_ref, ab2_ref, ab1_ref, ag_ref, abb_ref,
                 cw_ref, ci_ref, ew0_ref, ei0_ref, ew1_ref, ei1_ref, ad_ref,
                 W1s):
    i = pl.program_id(0)

    @pl.when(i == 0)
    def _():
        W1s[:, 0:H] = cW1_ref[...].astype(jnp.bfloat16)
        for c in range(C):
            W1s[:, H + c * H:H + (c + 1) * H] = eW1_ref[c].astype(jnp.bfloat16)
        W1s[:, H + C * H:H + C * H + AH] = aW1_ref[...].astype(jnp.bfloat16)

    xb = x_ref[...].astype(jnp.bfloat16)

    mm = jnp.dot(xb, W1s[...], preferred_element_type=jnp.float32)

    h = _gelu(_ln_f32(mm[:, :H] + cb1_ref[...], cg_ref[...], cbb_ref[...]))
    cl = jnp.dot(h.astype(jnp.bfloat16), cW2_ref[...],
                 preferred_element_type=jnp.float32) + cb2_ref[...]
    clT = jnp.transpose(cl, (1, 0))[:C]
    cp = _softmax_t(clT)
    cw1, ci1, cw2, ci2 = _top2_t(cp)

    ehb = []
    for c in range(C):
        seg = mm[:, H + c * H:H + (c + 1) * H]
        eh = _gelu(_ln_f32(seg + eb1_ref[c], eg_ref[c], ebb_ref[c]))
        ehb.append(eh.astype(jnp.bfloat16))
    ehb = jnp.concatenate(ehb, axis=1)
    el = jnp.dot(ehb, W2e_ref[...],
                 preferred_element_type=jnp.float32) + eb2_ref[...]
    elT = jnp.transpose(el, (1, 0))

    sel0 = jnp.full((_GRP, elT.shape[1]), _NEG, dtype=jnp.float32)
    sel1 = jnp.full((_GRP, elT.shape[1]), _NEG, dtype=jnp.float32)
    for c in range(C):
        grp = elT[c * _GRP:(c + 1) * _GRP]
        sel0 = jnp.where(ci1 == c, grp, sel0)
        sel1 = jnp.where(ci2 == c, grp, sel1)
    e0w1, e0i1, e0w2, e0i2 = _top2_t(_softmax_t(sel0))
    e1w1, e1i1, e1w2, e1i2 = _top2_t(_softmax_t(sel1))

    ah = _gelu(_ln_f32(mm[:, H + C * H:H + C * H + AH] + ab1_ref[...],
                       ag_ref[...], abb_ref[...]))
    av = jnp.dot(ah.astype(jnp.bfloat16), aW2_ref[...],
                 preferred_element_type=jnp.float32) + ab2_ref[...]
    ad = jax.nn.sigmoid(av[:, 0:1])

    cw_ref[...] = _pair_out(cw1, cw2)
    ci_ref[...] = _pair_out(ci1, ci2).astype(jnp.int32)
    ew0_ref[...] = _pair_out(e0w1, e0w2)
    ei0_ref[...] = _pair_out(e0i1, e0i2).astype(jnp.int32)
    ew1_ref[...] = _pair_out(e1w1, e1w2)
    ei1_ref[...] = _pair_out(e1i1, e1i2).astype(jnp.int32)
    ad_ref[...] = ad


def kernel(x, cW1, cb1, cg, cbb, cW2, cb2, eW1, eb1, eg, ebb, eW2, eb2,
           aW1, ab1, ag, abb, aW2, ab2):
    Bz, Sz, Dz = x.shape
    T = Bz * Sz
    C = eW1.shape[0]
    H = cW1.shape[1]
    AH = aW1.shape[1]
    EPC = eW2.shape[2]
    L = _LANES

    xf = x.reshape(T, Dz)

    cW2p = jnp.zeros((H, L), cW2.dtype).at[:, :C].set(cW2).astype(jnp.bfloat16)
    cb2p = jnp.full((L,), _NEG, jnp.float32).at[:C].set(cb2)
    W2e = jnp.zeros((C * H, L), eW2.dtype)
    eb2p = jnp.full((L,), _NEG, jnp.float32)
    for c in range(C):
        W2e = W2e.at[c * H:(c + 1) * H, c * _GRP:c * _GRP + EPC].set(eW2[c])
        eb2p = eb2p.at[c * _GRP:c * _GRP + EPC].set(eb2[c])
    W2e = W2e.astype(jnp.bfloat16)
    aW2p = jnp.zeros((AH, L), aW2.dtype).at[:, :1].set(aW2).astype(jnp.bfloat16)
    ab2p = jnp.zeros((L,), jnp.float32).at[:1].set(ab2)

    grid = (T // _TM,)
    tok_spec = pl.BlockSpec((_TM, Dz), lambda i: (i, 0))
    full = lambda *shape: pl.BlockSpec(shape, lambda i: (0,) * len(shape))
    out2 = pl.BlockSpec((_TM, 2), lambda i: (i, 0))
    out1 = pl.BlockSpec((_TM, 1), lambda i: (i, 0))

    f32 = jnp.float32
    i32 = jnp.int32
    outs = (
        jax.ShapeDtypeStruct((T, 2), f32), jax.ShapeDtypeStruct((T, 2), i32),
        jax.ShapeDtypeStruct((T, 2), f32), jax.ShapeDtypeStruct((T, 2), i32),
        jax.ShapeDtypeStruct((T, 2), f32), jax.ShapeDtypeStruct((T, 2), i32),
        jax.ShapeDtypeStruct((T, 1), f32),
    )

    cw, ci, ew0, ei0, ew1, ei1, ad = pl.pallas_call(
        functools.partial(_router_body, C, H, AH),
        grid=grid,
        in_specs=[
            tok_spec,
            full(Dz, H), full(C, Dz, H), full(Dz, AH),
            full(H), full(H, L), full(L), full(H), full(H),
            full(C * H, L), full(L), full(C, H), full(C, H), full(C, H),
            full(AH, L), full(L), full(AH), full(AH), full(AH),
        ],
        out_specs=[out2, out2, out2, out2, out2, out2, out1],
        out_shape=outs,
        scratch_shapes=[pltpu.VMEM((Dz, H + C * H + AH), jnp.bfloat16)],
        compiler_params=pltpu.CompilerParams(
            dimension_semantics=("arbitrary",),
        ),
    )(xf, cW1, eW1, aW1,
      cb1, cW2p, cb2p, cg, cbb,
      W2e, eb2p, eb1, eg, ebb,
      aW2p, ab2p, ab1, ag, abb)

    return (cw, ci, ew0, ei0, ew1, ei1, ad.reshape(Bz, Sz, 1))


# speedup vs baseline: 3.5852x; 1.2494x over previous
import functools

import jax
import jax.numpy as jnp
from jax.experimental import pallas as pl
from jax.experimental.pallas import tpu as pltpu

_TM = 512
_LANES = 128
_GRP = 16
_NEG = -1e30


def _gelu(h):
    return 0.5 * h * (1.0 + jax.lax.erf(h * 0.7071067811865476))


def _ln_f32(h, g, b):
    m = jnp.mean(h, axis=-1, keepdims=True)
    d = h - m
    v = jnp.mean(d * d, axis=-1, keepdims=True)
    return d * jax.lax.rsqrt(v + 1e-5) * g + b


def _top2_t(p):
    idx = jax.lax.broadcasted_iota(jnp.int32, p.shape, 0).astype(jnp.float32)
    w1 = jnp.max(p, axis=0, keepdims=True)
    i1 = jnp.min(jnp.where(p == w1, idx, jnp.float32(1e9)), axis=0, keepdims=True)
    p2 = jnp.where(idx == i1, _NEG, p)
    w2 = jnp.max(p2, axis=0, keepdims=True)
    i2 = jnp.min(jnp.where(p2 == w2, idx, jnp.float32(1e9)), axis=0, keepdims=True)
    return w1, i1, w2, i2


def _softmax_t(l):
    m = jnp.max(l, axis=0, keepdims=True)
    e = jnp.exp(l - m)
    return e / jnp.sum(e, axis=0, keepdims=True)


def _pair_out(a, b):
    return jnp.transpose(jnp.concatenate([a, b], axis=0), (1, 0))


def _router_body(C, H, AH,
                 x_ref, cW1_ref, eW1_ref, aW1_ref,
                 cb1_ref, cW2_ref, cb2_ref, cg_ref, cbb_ref,
                 eW2_ref, eb2_ref, eb1_ref, eg_ref, ebb_ref,
                 aW2_ref, ab2_ref, ab1_ref, ag_ref, abb_ref,
                 cw_ref, ci_ref, ew0_ref, ei0_ref, ew1_ref, ei1_ref, ad_ref,
                 W1s, cW2s, W2es, aW2s, b2s):
    i = pl.program_id(0)
    L = _LANES
    f32 = jnp.float32
    bf16 = jnp.bfloat16

    @pl.when(i == 0)
    def _():
        W1s[:, 0:H] = cW1_ref[...].astype(bf16)
        for c in range(C):
            W1s[:, H + c * H:H + (c + 1) * H] = eW1_ref[c].astype(bf16)
        W1s[:, H + C * H:H + C * H + AH] = aW1_ref[...].astype(bf16)

        nc = cW2_ref.shape[1]
        cW2s[...] = jnp.concatenate(
            [cW2_ref[...].astype(bf16), jnp.zeros((H, L - nc), bf16)], axis=1)
        ne = eW2_ref.shape[2]
        eW2f = eW2_ref[...]
        for c in range(C):
            parts = []
            if c > 0:
                parts.append(jnp.zeros((H, _GRP * c), bf16))
            parts.append(eW2f[c].astype(bf16))
            parts.append(jnp.zeros((H, L - _GRP * c - ne), bf16))
            W2es[c * H:(c + 1) * H, :] = jnp.concatenate(parts, axis=1)
        aW2s[...] = jnp.concatenate(
            [aW2_ref[...].astype(bf16), jnp.zeros((AH, L - 1), bf16)], axis=1)

        b2s[0:1, :] = jnp.concatenate(
            [cb2_ref[...], jnp.full((1, L - nc), _NEG, f32)], axis=1)
        eb2f = eb2_ref[...]
        erow = jnp.full((1, L), _NEG, f32)
        lane = jax.lax.broadcasted_iota(jnp.int32, (1, L), 1)
        for c in range(C):
            parts = []
            if c > 0:
                parts.append(jnp.zeros((1, _GRP * c), f32))
            parts.append(eb2f[c:c + 1, :])
            parts.append(jnp.zeros((1, L - _GRP * c - ne), f32))
            row = jnp.concatenate(parts, axis=1)
            m = (lane >= _GRP * c) & (lane < _GRP * c + ne)
            erow = jnp.where(m, row, erow)
        b2s[1:2, :] = erow
        b2s[2:3, :] = jnp.concatenate(
            [ab2_ref[...], jnp.zeros((1, L - 1), f32)], axis=1)

    xb = x_ref[...].astype(bf16)

    mm = jnp.dot(xb, W1s[...], preferred_element_type=jnp.float32)

    h = _gelu(_ln_f32(mm[:, :H] + cb1_ref[...], cg_ref[...], cbb_ref[...]))
    cl = jnp.dot(h.astype(jnp.bfloat16), cW2s[...],
                 preferred_element_type=jnp.float32) + b2s[0:1, :]
    clT = jnp.transpose(cl, (1, 0))[:C]
    cp = _softmax_t(clT)
    cw1, ci1, cw2, ci2 = _top2_t(cp)

    ehb = []
    for c in range(C):
        seg = mm[:, H + c * H:H + (c + 1) * H]
        eh = _gelu(_ln_f32(seg + eb1_ref[c], eg_ref[c], ebb_ref[c]))
        ehb.append(eh.astype(jnp.bfloat16))
    ehb = jnp.concatenate(ehb, axis=1)
    el = jnp.dot(ehb, W2es[...],
                 preferred_element_type=jnp.float32) + b2s[1:2, :]
    elT = jnp.transpose(el, (1, 0))

    sel0 = jnp.full((_GRP, elT.shape[1]), _NEG, dtype=jnp.float32)
    sel1 = jnp.full((_GRP, elT.shape[1]), _NEG, dtype=jnp.float32)
    for c in range(C):
        grp = elT[c * _GRP:(c + 1) * _GRP]
        sel0 = jnp.where(ci1 == c, grp, sel0)
        sel1 = jnp.where(ci2 == c, grp, sel1)
    e0w1, e0i1, e0w2, e0i2 = _top2_t(_softmax_t(sel0))
    e1w1, e1i1, e1w2, e1i2 = _top2_t(_softmax_t(sel1))

    ah = _gelu(_ln_f32(mm[:, H + C * H:H + C * H + AH] + ab1_ref[...],
                       ag_ref[...], abb_ref[...]))
    av = jnp.dot(ah.astype(jnp.bfloat16), aW2s[...],
                 preferred_element_type=jnp.float32) + b2s[2:3, :]
    ad = jax.nn.sigmoid(av[:, 0:1])

    cw_ref[...] = _pair_out(cw1, cw2)
    ci_ref[...] = _pair_out(ci1, ci2).astype(jnp.int32)
    ew0_ref[...] = _pair_out(e0w1, e0w2)
    ei0_ref[...] = _pair_out(e0i1, e0i2).astype(jnp.int32)
    ew1_ref[...] = _pair_out(e1w1, e1w2)
    ei1_ref[...] = _pair_out(e1i1, e1i2).astype(jnp.int32)
    ad_ref[...] = ad


def kernel(x, cW1, cb1, cg, cbb, cW2, cb2, eW1, eb1, eg, ebb, eW2, eb2,
           aW1, ab1, ag, abb, aW2, ab2):
    Bz, Sz, Dz = x.shape
    T = Bz * Sz
    C = eW1.shape[0]
    H = cW1.shape[1]
    AH = aW1.shape[1]
    EPC = eW2.shape[2]
    L = _LANES

    xf = x.reshape(T, Dz)
    cb2r = cb2.reshape(1, C)
    ab2r = ab2.reshape(1, 1)

    grid = (T // _TM,)
    tok_spec = pl.BlockSpec((_TM, Dz), lambda i: (i, 0))
    full = lambda *shape: pl.BlockSpec(shape, lambda i: (0,) * len(shape))
    out2 = pl.BlockSpec((_TM, 2), lambda i: (i, 0))
    out1 = pl.BlockSpec((_TM, 1), lambda i: (i, 0))

    f32 = jnp.float32
    i32 = jnp.int32
    outs = (
        jax.ShapeDtypeStruct((T, 2), f32), jax.ShapeDtypeStruct((T, 2), i32),
        jax.ShapeDtypeStruct((T, 2), f32), jax.ShapeDtypeStruct((T, 2), i32),
        jax.ShapeDtypeStruct((T, 2), f32), jax.ShapeDtypeStruct((T, 2), i32),
        jax.ShapeDtypeStruct((T, 1), f32),
    )

    cw, ci, ew0, ei0, ew1, ei1, ad = pl.pallas_call(
        functools.partial(_router_body, C, H, AH),
        grid=grid,
        in_specs=[
            tok_spec,
            full(Dz, H), full(C, Dz, H), full(Dz, AH),
            full(H), full(H, C), full(1, C), full(H), full(H),
            full(C, H, EPC), full(C, EPC), full(C, H), full(C, H), full(C, H),
            full(AH, 1), full(1, 1), full(AH), full(AH), full(AH),
        ],
        out_specs=[out2, out2, out2, out2, out2, out2, out1],
        out_shape=outs,
        scratch_shapes=[
            pltpu.VMEM((Dz, H + C * H + AH), jnp.bfloat16),
            pltpu.VMEM((H, L), jnp.bfloat16),
            pltpu.VMEM((C * H, L), jnp.bfloat16),
            pltpu.VMEM((AH, L), jnp.bfloat16),
            pltpu.VMEM((8, L), jnp.float32),
        ],
        compiler_params=pltpu.CompilerParams(
            dimension_semantics=("arbitrary",),
        ),
    )(xf, cW1, eW1, aW1,
      cb1, cW2, cb2r, cg, cbb,
      eW2, eb2, eb1, eg, ebb,
      aW2, ab2r, ab1, ag, abb)

    return (cw, ci, ew0, ei0, ew1, ei1, ad.reshape(Bz, Sz, 1))
